# baseline (speedup 1.0000x reference)
import functools

import jax
import jax.numpy as jnp
from jax import lax
from jax.experimental import pallas as pl
from jax.experimental.pallas import tpu as pltpu
from jax.experimental.pallas import tpu_sc as plsc

N = 16384
P = 16384
K = 64
NNZ = 268435

NC = 2
NS = 16
L = 16
NW = NC * NS
BATCH = 128
NB0 = 72
NB1 = 60
NBMAX = max(NB0, NB1)
NPAD = NS * (NB0 + NB1) * BATCH
NB = NBMAX
XB = 32
RPT = 16384 // NS

PB = 512
GRID = 16384 // PB


def _make_sc_pass(do_x):
  ring = 2 if do_x else 3
  mesh = plsc.VectorSubcoreMesh(core_axis_name="c", subcore_axis_name="s",
                                num_cores=NC, num_subcores=NS)
  outs = [jax.ShapeDtypeStruct((NC, 16384, K), jnp.float32)]
  scratches = [
      pltpu.VMEM_SHARED((16384, K), jnp.float32),
      pltpu.VMEM((NB, BATCH), jnp.int32),
      pltpu.VMEM((NB, BATCH), jnp.int32),
      pltpu.VMEM((NB, BATCH), jnp.float32),
      pltpu.VMEM((ring, BATCH, K), jnp.float32),
      pltpu.VMEM((BATCH, K), jnp.float32),
  ] + [pltpu.SemaphoreType.DMA] * (2 * ring)
  if do_x:
    outs += [
        jax.ShapeDtypeStruct((NC, P), jnp.float32),
        jax.ShapeDtypeStruct((NC, 16384, K), jnp.float32),
    ]
    scratches += [
        pltpu.VMEM_SHARED((P,), jnp.float32),
        pltpu.VMEM((1024,), jnp.float32),
        pltpu.VMEM((BATCH,), jnp.float32),
        pltpu.VMEM((3, XB, K), jnp.float32),
        pltpu.VMEM((K,), jnp.float32),
    ]

  @functools.partial(pl.kernel, out_type=outs, mesh=mesh,
                     scratch_types=scratches,
                     compiler_params=pltpu.CompilerParams(
                         use_tc_tiling_on_sc=False))
  def body(vals_h, gidx_h, sidx_h, *rest):
    if do_x:
      mz_h, pp_h, inter_h = rest[0:3]
      acc_out, gsq_out, x_out = rest[3:6]
      acc_sh, gi, si, va, gb, z2 = rest[6:12]
      gsems = rest[12:12 + ring]
      ssems = rest[12 + ring:12 + 2 * ring]
      gsq_sh, z1, sq, xin, ibuf = rest[12 + 2 * ring:17 + 2 * ring]
      table_h = None
    else:
      table_h = rest[0]
      acc_out = rest[1]
      acc_sh, gi, si, va, gb, z2 = rest[2:8]
      gsems, ssems = rest[8:8 + ring], rest[8 + ring:8 + 2 * ring]
      gsq_out = gsq_sh = z1 = sq = xin = ibuf = x_out = None
    c = lax.axis_index("c")
    s = lax.axis_index("s")
    tid = c * NS + s
    nq = jnp.where(c == 0, NB0 // ring, NB1 // ring)
    nb = nq * ring
    if do_x:
      table_h = x_out.at[c]
    do_gsq = do_x

    pltpu.sync_copy(gidx_h.at[tid], gi)
    pltpu.sync_copy(sidx_h.at[tid], si)
    pltpu.sync_copy(vals_h.at[tid], va)

    if do_x:
      pltpu.sync_copy(inter_h.at[0], ibuf)
      base0 = s * RPT

      def xchunk(m, _):
        b = base0 + m * XB
        pltpu.sync_copy(pp_h.at[:, pl.ds(b, XB)], xin.at[pl.ds(0, 2)])
        pltpu.sync_copy(mz_h.at[pl.ds(b, XB)], xin.at[2])

        @plsc.parallel_loop(0, XB // L)
        def _(ci):
          for r in range(L):
            i = ci * L + r
            for jj in range(K // L):
              iv = ibuf[pl.ds(jj * L, L)]
              x = (xin[2, i, pl.ds(jj * L, L)]
                   - xin[0, i, pl.ds(jj * L, L)]
                   - xin[1, i, pl.ds(jj * L, L)] - iv)
              xin[2, i, pl.ds(jj * L, L)] = x
        pltpu.sync_copy(xin.at[2], x_out.at[c].at[pl.ds(b, XB)])
        return 0
      lax.fori_loop(0, RPT // XB, xchunk, 0)

    def zrow(i, _):
      for j in range(K // L):
        z2[i, pl.ds(j * L, L)] = jnp.zeros((L,), jnp.float32)
      return 0
    lax.fori_loop(0, BATCH, zrow, 0)
    for r in range(RPT // BATCH):
      pltpu.sync_copy(z2, acc_sh.at[pl.ds(s * RPT + r * BATCH, BATCH)])
    if do_gsq:
      def z1row(i, _):
        z1[pl.ds(i * L, L)] = jnp.zeros((L,), jnp.float32)
        return 0
      lax.fori_loop(0, 1024 // L, z1row, 0)
      pltpu.sync_copy(z1, gsq_sh.at[pl.ds(s * 1024, 1024)])
    plsc.subcore_barrier()

    def fire_gather(g, j):
      pltpu.async_copy(table_h.at[gi.at[g]], gb.at[j], gsems[j])

    def wait_gather(g, j):
      pltpu.make_async_copy(table_h.at[gi.at[g]], gb.at[j], gsems[j]).wait()

    def fire_scatter(g, j):
      pltpu.async_copy(gb.at[j], acc_sh.at[si.at[g]], ssems[j], add=True)

    def wait_scatter(g, j):
      pltpu.make_async_copy(gb.at[j], acc_sh.at[si.at[g]], ssems[j]).wait()

    def scale(g, j):
      @plsc.parallel_loop(0, BATCH // L)
      def _(cidx):
        vc = va[g, pl.ds(cidx * L, L)]
        for r in range(L):
          vb = lax.gather(
              vc, jnp.full((L, 1), r, jnp.int32),
              lax.GatherDimensionNumbers(offset_dims=(),
                                         collapsed_slice_dims=(0,),
                                         start_index_map=(0,)),
              (1,), mode=lax.GatherScatterMode.PROMISE_IN_BOUNDS)
          i = cidx * L + r
          for jj in range(K // L):
            x = gb[j, i, pl.ds(jj * L, L)]
            gb[j, i, pl.ds(jj * L, L)] = x * vb
      if do_gsq:
        def sq_grp(jj, _):
          v = va[g, pl.ds(jj * L, L)]
          sq[pl.ds(jj * L, L)] = v * v
          return 0
        lax.fori_loop(0, BATCH // L, sq_grp, 0)
        pltpu.sync_copy(sq, gsq_sh.at[si.at[g]], add=True)

    for j in range(ring - 1):
      fire_gather(j, j)

    def quad(q, _):
      for r in range(ring):
        g = q * ring + r
        wait_gather(g, r)
        scale(g, r)
        fire_scatter(g, r)
        j3 = (r + ring - 1) % ring
        if r == 0:
          @pl.when(q > 0)
          def _():
            wait_scatter(g - 1, j3)
          fire_gather(g + ring - 1, j3)
        else:
          @pl.when(q < nq - 1)
          def _():
            wait_scatter(g - 1, j3)
            fire_gather(g + ring - 1, j3)
      return 0
    lax.fori_loop(0, nq, quad, 0)

    for j in range(ring):
      wait_scatter(nb - ring + j, j)

    plsc.subcore_barrier()
    pltpu.sync_copy(acc_sh.at[pl.ds(s * RPT, RPT)],
                    acc_out.at[c].at[pl.ds(s * RPT, RPT)])
    if do_gsq:
      pltpu.sync_copy(gsq_sh.at[pl.ds(s * 1024, 1024)],
                      gsq_out.at[c].at[pl.ds(s * 1024, 1024)])

  return body


@functools.lru_cache(maxsize=None)
def _sc_pass(do_gsq):
  return _make_sc_pass(do_gsq)


def _sc_pred(vals, gidx, sidx, table):
  out = _sc_pass(False)(vals, gidx, sidx, table)
  return out[0] if isinstance(out, (list, tuple)) else out


def _sc_xtg(vals, gidx, sidx, mean_z, pred_part, inter):
  acc, gsq, _ = _sc_pass(True)(vals, gidx, sidx, mean_z, pred_part, inter)
  return acc, gsq


def _prep_body(mb_ref, ph_ref, mz_ref, b_ref, inter_ref):
  i = pl.program_id(0)
  b_ref[...] = mb_ref[...] * ph_ref[...].T
  part = jnp.sum(mz_ref[...], axis=0, keepdims=True) * (1.0 / N)

  @pl.when(i == 0)
  def _():
    inter_ref[...] = part

  @pl.when(i > 0)
  def _():
    inter_ref[...] += part


def _prep(mean_beta, p_hat, mean_z):
  return pl.pallas_call(
      _prep_body,
      grid=(GRID,),
      in_specs=[
          pl.BlockSpec((PB, K), lambda i: (i, 0)),
          pl.BlockSpec((K, PB), lambda i: (0, i)),
          pl.BlockSpec((PB, K), lambda i: (i, 0)),
      ],
      out_specs=[
          pl.BlockSpec((PB, K), lambda i: (i, 0)),
          pl.BlockSpec((1, K), lambda i: (0, 0)),
      ],
      out_shape=[
          jax.ShapeDtypeStruct((P, K), jnp.float32),
          jax.ShapeDtypeStruct((1, K), jnp.float32),
      ],
  )(mean_beta, p_hat, mean_z)


def _final_body(xtg_ref, b_ref, gsq_ref, tau_ref, lp_ref,
                mb_ref, vb_ref, ph_ref):
  gsq = (gsq_ref[0] + gsq_ref[1])[:, None]
  zkg = xtg_ref[0] + xtg_ref[1] + b_ref[...] * gsq
  var = 1.0 / (tau_ref[...] + gsq)
  mb = zkg * var
  ph = 1.0 / (1.0 + jnp.exp(-(lp_ref[0, 0] + 0.5 * mb * mb / var)))
  ph = jnp.clip(ph, 1e-8, 1.0 - 1e-8)
  mb_ref[...] = mb
  vb_ref[...] = var
  ph_ref[...] = ph.T


def _final(xtg_part, bmat, gsq_part, tau, lp):
  return pl.pallas_call(
      _final_body,
      grid=(GRID,),
      in_specs=[
          pl.BlockSpec((NC, PB, K), lambda i: (0, i, 0)),
          pl.BlockSpec((PB, K), lambda i: (i, 0)),
          pl.BlockSpec((NC, PB), lambda i: (0, i)),
          pl.BlockSpec((1, K), lambda i: (0, 0)),
          pl.BlockSpec((1, 1), lambda i: (0, 0)),
      ],
      out_specs=[
          pl.BlockSpec((PB, K), lambda i: (i, 0)),
          pl.BlockSpec((PB, K), lambda i: (i, 0)),
          pl.BlockSpec((K, PB), lambda i: (0, i)),
      ],
      out_shape=[
          jax.ShapeDtypeStruct((P, K), jnp.float32),
          jax.ShapeDtypeStruct((P, K), jnp.float32),
          jax.ShapeDtypeStruct((K, P), jnp.float32),
      ],
  )(xtg_part, bmat, gsq_part, tau, lp)


def kernel(guide_values, guide_rows, guide_cols, mean_z, mean_beta,
           var_beta, p_hat, tau_beta, p):
  pad = NPAD - NNZ

  def _shard(x):
    e0 = NS * NB0 * BATCH
    a = x[:e0].reshape(NS, NB0, BATCH)
    a = jnp.pad(a, ((0, 0), (0, NBMAX - NB0), (0, 0)))
    b = x[e0:].reshape(NS, NB1, BATCH)
    b = jnp.pad(b, ((0, 0), (0, NBMAX - NB1), (0, 0)))
    return jnp.concatenate([a, b], axis=0)

  vals = _shard(jnp.concatenate([guide_values, jnp.zeros((pad,),
                                                         jnp.float32)]))
  rows = _shard(jnp.concatenate([guide_rows,
                                 jnp.zeros((pad,), guide_rows.dtype)]))
  cols = _shard(jnp.concatenate([guide_cols,
                                 jnp.zeros((pad,), guide_cols.dtype)]))

  bmat, inter = _prep(mean_beta, p_hat, mean_z)
  pred_part = _sc_pred(vals, cols, rows, bmat)
  xtg_part, gsq_part = _sc_xtg(vals, rows, cols, mean_z, pred_part, inter)

  lp = (jnp.log(p) - jnp.log1p(-p)).reshape(1, 1).astype(jnp.float32)
  tau = tau_beta.reshape(1, K)
  return _final(xtg_part, bmat, gsq_part, tau, lp)


# speedup vs baseline: 1.1804x; 1.1804x over previous
import functools

import jax
import jax.numpy as jnp
from jax import lax
from jax.experimental import pallas as pl
from jax.experimental.pallas import tpu as pltpu
from jax.experimental.pallas import tpu_sc as plsc

N = 16384
P = 16384
K = 64
NNZ = 268435

NC = 2
NS = 16
L = 16
NW = NC * NS
BATCH = 128
NB0 = 72
NB1 = 60
NBMAX = max(NB0, NB1)
NPAD = NS * (NB0 + NB1) * BATCH
NB = NBMAX
XB = 32
RPT = 16384 // NS

PB = 512
GRID = 16384 // PB


def _make_sc_pass(do_x):
  ring = 3
  mesh = plsc.VectorSubcoreMesh(core_axis_name="c", subcore_axis_name="s",
                                num_cores=NC, num_subcores=NS)
  outs = [jax.ShapeDtypeStruct((NC, 16384, K), jnp.float32)]
  scratches = [
      pltpu.VMEM_SHARED((16384, K), jnp.float32),
      pltpu.VMEM((NB, BATCH), jnp.int32),
      pltpu.VMEM((NB, BATCH), jnp.int32),
      pltpu.VMEM((NB, BATCH), jnp.float32),
      pltpu.VMEM((ring, BATCH, K), jnp.float32),
      pltpu.VMEM((BATCH, K), jnp.float32),
  ] + [pltpu.SemaphoreType.DMA] * (2 * ring)
  if do_x:
    outs += [
        jax.ShapeDtypeStruct((NC, P), jnp.float32),
    ]
    scratches += [
        pltpu.VMEM_SHARED((P,), jnp.float32),
        pltpu.VMEM((1024,), jnp.float32),
        pltpu.VMEM((BATCH,), jnp.float32),
    ]

  @functools.partial(pl.kernel, out_type=outs, mesh=mesh,
                     scratch_types=scratches,
                     compiler_params=pltpu.CompilerParams(
                         use_tc_tiling_on_sc=False))
  def body(vals_h, gidx_h, sidx_h, *rest):
    table_h = rest[0]
    if do_x:
      acc_out, gsq_out = rest[1:3]
      acc_sh, gi, si, va, gb, z2 = rest[3:9]
      gsems, ssems = rest[9:9 + ring], rest[9 + ring:9 + 2 * ring]
      gsq_sh, z1, sq = rest[9 + 2 * ring:12 + 2 * ring]
    else:
      acc_out = rest[1]
      acc_sh, gi, si, va, gb, z2 = rest[2:8]
      gsems, ssems = rest[8:8 + ring], rest[8 + ring:8 + 2 * ring]
      gsq_out = gsq_sh = z1 = sq = None
    c = lax.axis_index("c")
    s = lax.axis_index("s")
    tid = c * NS + s
    nq = jnp.where(c == 0, NB0 // ring, NB1 // ring)
    nb = nq * ring
    do_gsq = do_x

    pltpu.sync_copy(gidx_h.at[tid], gi)
    pltpu.sync_copy(sidx_h.at[tid], si)
    pltpu.sync_copy(vals_h.at[tid], va)

    def zrow(i, _):
      for j in range(K // L):
        z2[i, pl.ds(j * L, L)] = jnp.zeros((L,), jnp.float32)
      return 0
    lax.fori_loop(0, BATCH, zrow, 0)
    for r in range(RPT // BATCH):
      pltpu.sync_copy(z2, acc_sh.at[pl.ds(s * RPT + r * BATCH, BATCH)])
    if do_gsq:
      def z1row(i, _):
        z1[pl.ds(i * L, L)] = jnp.zeros((L,), jnp.float32)
        return 0
      lax.fori_loop(0, 1024 // L, z1row, 0)
      pltpu.sync_copy(z1, gsq_sh.at[pl.ds(s * 1024, 1024)])
    plsc.subcore_barrier()

    def fire_gather(g, j):
      pltpu.async_copy(table_h.at[gi.at[g]], gb.at[j], gsems[j])

    def wait_gather(g, j):
      pltpu.make_async_copy(table_h.at[gi.at[g]], gb.at[j], gsems[j]).wait()

    def fire_scatter(g, j):
      pltpu.async_copy(gb.at[j], acc_sh.at[si.at[g]], ssems[j], add=True)

    def wait_scatter(g, j):
      pltpu.make_async_copy(gb.at[j], acc_sh.at[si.at[g]], ssems[j]).wait()

    def scale(g, j):
      @plsc.parallel_loop(0, BATCH // L)
      def _(cidx):
        vc = va[g, pl.ds(cidx * L, L)]
        for r in range(L):
          vb = lax.gather(
              vc, jnp.full((L, 1), r, jnp.int32),
              lax.GatherDimensionNumbers(offset_dims=(),
                                         collapsed_slice_dims=(0,),
                                         start_index_map=(0,)),
              (1,), mode=lax.GatherScatterMode.PROMISE_IN_BOUNDS)
          i = cidx * L + r
          for jj in range(K // L):
            x = gb[j, i, pl.ds(jj * L, L)]
            gb[j, i, pl.ds(jj * L, L)] = x * vb
      if do_gsq:
        def sq_grp(jj, _):
          v = va[g, pl.ds(jj * L, L)]
          sq[pl.ds(jj * L, L)] = v * v
          return 0
        lax.fori_loop(0, BATCH // L, sq_grp, 0)
        pltpu.sync_copy(sq, gsq_sh.at[si.at[g]], add=True)

    for j in range(ring - 1):
      fire_gather(j, j)

    def quad(q, _):
      for r in range(ring):
        g = q * ring + r
        wait_gather(g, r)
        scale(g, r)
        fire_scatter(g, r)
        j3 = (r + ring - 1) % ring
        if r == 0:
          @pl.when(q > 0)
          def _():
            wait_scatter(g - 1, j3)
          fire_gather(g + ring - 1, j3)
        else:
          @pl.when(q < nq - 1)
          def _():
            wait_scatter(g - 1, j3)
            fire_gather(g + ring - 1, j3)
      return 0
    lax.fori_loop(0, nq, quad, 0)

    for j in range(ring):
      wait_scatter(nb - ring + j, j)

    plsc.subcore_barrier()
    pltpu.sync_copy(acc_sh.at[pl.ds(s * RPT, RPT)],
                    acc_out.at[c].at[pl.ds(s * RPT, RPT)])
    if do_gsq:
      pltpu.sync_copy(gsq_sh.at[pl.ds(s * 1024, 1024)],
                      gsq_out.at[c].at[pl.ds(s * 1024, 1024)])

  return body


@functools.lru_cache(maxsize=None)
def _sc_pass(do_gsq):
  return _make_sc_pass(do_gsq)


def _sc_pred(vals, gidx, sidx, table):
  out = _sc_pass(False)(vals, gidx, sidx, table)
  return out[0] if isinstance(out, (list, tuple)) else out


def _sc_xtg(vals, gidx, sidx, table):
  acc, gsq = _sc_pass(True)(vals, gidx, sidx, table)
  return acc, gsq


def _prep_body(mb_ref, ph_ref, mz_ref, b_ref, inter_ref):
  i = pl.program_id(0)
  b_ref[...] = mb_ref[...] * ph_ref[...].T
  part = jnp.sum(mz_ref[...], axis=0, keepdims=True) * (1.0 / N)

  @pl.when(i == 0)
  def _():
    inter_ref[...] = part

  @pl.when(i > 0)
  def _():
    inter_ref[...] += part


def _prep(mean_beta, p_hat, mean_z):
  return pl.pallas_call(
      _prep_body,
      grid=(GRID,),
      in_specs=[
          pl.BlockSpec((PB, K), lambda i: (i, 0)),
          pl.BlockSpec((K, PB), lambda i: (0, i)),
          pl.BlockSpec((PB, K), lambda i: (i, 0)),
      ],
      out_specs=[
          pl.BlockSpec((PB, K), lambda i: (i, 0)),
          pl.BlockSpec((1, K), lambda i: (0, 0)),
      ],
      out_shape=[
          jax.ShapeDtypeStruct((P, K), jnp.float32),
          jax.ShapeDtypeStruct((1, K), jnp.float32),
      ],
  )(mean_beta, p_hat, mean_z)


def _x_body(mz_ref, pp_ref, inter_ref, x_ref):
  x_ref[...] = mz_ref[...] - pp_ref[0] - pp_ref[1] - inter_ref[...]


def _xcalc(mean_z, pred_part, inter):
  return pl.pallas_call(
      _x_body,
      grid=(GRID,),
      in_specs=[
          pl.BlockSpec((PB, K), lambda i: (i, 0)),
          pl.BlockSpec((NC, PB, K), lambda i: (0, i, 0)),
          pl.BlockSpec((1, K), lambda i: (0, 0)),
      ],
      out_specs=pl.BlockSpec((PB, K), lambda i: (i, 0)),
      out_shape=jax.ShapeDtypeStruct((N, K), jnp.float32),
  )(mean_z, pred_part, inter)


def _final_body(xtg_ref, b_ref, gsq_ref, tau_ref, lp_ref,
                mb_ref, vb_ref, ph_ref):
  gsq = (gsq_ref[0] + gsq_ref[1])[:, None]
  zkg = xtg_ref[0] + xtg_ref[1] + b_ref[...] * gsq
  var = 1.0 / (tau_ref[...] + gsq)
  mb = zkg * var
  ph = 1.0 / (1.0 + jnp.exp(-(lp_ref[0, 0] + 0.5 * mb * mb / var)))
  ph = jnp.clip(ph, 1e-8, 1.0 - 1e-8)
  mb_ref[...] = mb
  vb_ref[...] = var
  ph_ref[...] = ph.T


def _final(xtg_part, bmat, gsq_part, tau, lp):
  return pl.pallas_call(
      _final_body,
      grid=(GRID,),
      in_specs=[
          pl.BlockSpec((NC, PB, K), lambda i: (0, i, 0)),
          pl.BlockSpec((PB, K), lambda i: (i, 0)),
          pl.BlockSpec((NC, PB), lambda i: (0, i)),
          pl.BlockSpec((1, K), lambda i: (0, 0)),
          pl.BlockSpec((1, 1), lambda i: (0, 0)),
      ],
      out_specs=[
          pl.BlockSpec((PB, K), lambda i: (i, 0)),
          pl.BlockSpec((PB, K), lambda i: (i, 0)),
          pl.BlockSpec((K, PB), lambda i: (0, i)),
      ],
      out_shape=[
          jax.ShapeDtypeStruct((P, K), jnp.float32),
          jax.ShapeDtypeStruct((P, K), jnp.float32),
          jax.ShapeDtypeStruct((K, P), jnp.float32),
      ],
  )(xtg_part, bmat, gsq_part, tau, lp)


def kernel(guide_values, guide_rows, guide_cols, mean_z, mean_beta,
           var_beta, p_hat, tau_beta, p):
  pad = NPAD - NNZ

  def _shard(x):
    e0 = NS * NB0 * BATCH
    a = x[:e0].reshape(NS, NB0, BATCH)
    a = jnp.pad(a, ((0, 0), (0, NBMAX - NB0), (0, 0)))
    b = x[e0:].reshape(NS, NB1, BATCH)
    b = jnp.pad(b, ((0, 0), (0, NBMAX - NB1), (0, 0)))
    return jnp.concatenate([a, b], axis=0)

  vals = _shard(jnp.concatenate([guide_values, jnp.zeros((pad,),
                                                         jnp.float32)]))
  rows = _shard(jnp.concatenate([guide_rows,
                                 jnp.zeros((pad,), guide_rows.dtype)]))
  cols = _shard(jnp.concatenate([guide_cols,
                                 jnp.zeros((pad,), guide_cols.dtype)]))

  bmat, inter = _prep(mean_beta, p_hat, mean_z)
  pred_part = _sc_pred(vals, cols, rows, bmat)
  x = _xcalc(mean_z, pred_part, inter)
  xtg_part, gsq_part = _sc_xtg(vals, rows, cols, x)

  lp = (jnp.log(p) - jnp.log1p(-p)).reshape(1, 1).astype(jnp.float32)
  tau = tau_beta.reshape(1, K)
  return _final(xtg_part, bmat, gsq_part, tau, lp)


# speedup vs baseline: 1.1987x; 1.0155x over previous
import functools

import jax
import jax.numpy as jnp
from jax import lax
from jax.experimental import pallas as pl
from jax.experimental.pallas import tpu as pltpu
from jax.experimental.pallas import tpu_sc as plsc

N = 16384
P = 16384
K = 64
NNZ = 268435

NC = 2
NS = 16
L = 16
NW = NC * NS
BATCH = 128
NB0 = 72
NB1 = 60
NBMAX = max(NB0, NB1)
NPAD = NS * (NB0 + NB1) * BATCH
NB = NBMAX
XB = 32
RPT = 16384 // NS

PB = 512
GRID = 16384 // PB


def _make_sc_pass(do_x):
  ring = 3
  mesh = plsc.VectorSubcoreMesh(core_axis_name="c", subcore_axis_name="s",
                                num_cores=NC, num_subcores=NS)
  outs = [jax.ShapeDtypeStruct((NC, 16384, K), jnp.float32)]
  scratches = [
      pltpu.VMEM_SHARED((16384, K), jnp.float32),
      pltpu.VMEM((NB, BATCH), jnp.int32),
      pltpu.VMEM((NB, BATCH), jnp.int32),
      pltpu.VMEM((NB, BATCH), jnp.float32),
      pltpu.VMEM((ring, BATCH, K), jnp.float32),
      pltpu.VMEM((BATCH, K), jnp.float32),
  ] + [pltpu.SemaphoreType.DMA] * (2 * ring)
  if do_x:
    outs += [
        jax.ShapeDtypeStruct((NC, P), jnp.float32),
    ]
    scratches += [
        pltpu.VMEM_SHARED((P,), jnp.float32),
        pltpu.VMEM((1024,), jnp.float32),
        pltpu.VMEM((BATCH,), jnp.float32),
    ]

  @functools.partial(pl.kernel, out_type=outs, mesh=mesh,
                     scratch_types=scratches,
                     compiler_params=pltpu.CompilerParams(
                         use_tc_tiling_on_sc=False))
  def body(vals_h, gidx_h, sidx_h, *rest):
    table_h = rest[0]
    if do_x:
      acc_out, gsq_out = rest[1:3]
      acc_sh, gi, si, va, gb, z2 = rest[3:9]
      gsems, ssems = rest[9:9 + ring], rest[9 + ring:9 + 2 * ring]
      gsq_sh, z1, sq = rest[9 + 2 * ring:12 + 2 * ring]
    else:
      acc_out = rest[1]
      acc_sh, gi, si, va, gb, z2 = rest[2:8]
      gsems, ssems = rest[8:8 + ring], rest[8 + ring:8 + 2 * ring]
      gsq_out = gsq_sh = z1 = sq = None
    c = lax.axis_index("c")
    s = lax.axis_index("s")
    tid = c * NS + s
    nq = jnp.where(c == 0, NB0 // ring, NB1 // ring)
    nb = nq * ring
    do_gsq = do_x

    pltpu.sync_copy(gidx_h.at[tid], gi)
    pltpu.sync_copy(sidx_h.at[tid], si)
    pltpu.sync_copy(vals_h.at[tid], va)

    def zrow(i, _):
      for j in range(K // L):
        z2[i, pl.ds(j * L, L)] = jnp.zeros((L,), jnp.float32)
      return 0
    lax.fori_loop(0, BATCH, zrow, 0)
    for r in range(RPT // BATCH):
      pltpu.sync_copy(z2, acc_sh.at[pl.ds(s * RPT + r * BATCH, BATCH)])
    if do_gsq:
      def z1row(i, _):
        z1[pl.ds(i * L, L)] = jnp.zeros((L,), jnp.float32)
        return 0
      lax.fori_loop(0, 1024 // L, z1row, 0)
      pltpu.sync_copy(z1, gsq_sh.at[pl.ds(s * 1024, 1024)])
    plsc.subcore_barrier()

    def fire_gather(g, j):
      pltpu.async_copy(table_h.at[gi.at[g]], gb.at[j], gsems[j])

    def wait_gather(g, j):
      pltpu.make_async_copy(table_h.at[gi.at[g]], gb.at[j], gsems[j]).wait()

    def fire_scatter(g, j):
      pltpu.async_copy(gb.at[j], acc_sh.at[si.at[g]], ssems[j], add=True)

    def wait_scatter(g, j):
      pltpu.make_async_copy(gb.at[j], acc_sh.at[si.at[g]], ssems[j]).wait()

    def scale(g, j):
      @plsc.parallel_loop(0, BATCH // L, unroll=2)
      def _(cidx):
        vc = va[g, pl.ds(cidx * L, L)]
        for r in range(L):
          vb = lax.gather(
              vc, jnp.full((L, 1), r, jnp.int32),
              lax.GatherDimensionNumbers(offset_dims=(),
                                         collapsed_slice_dims=(0,),
                                         start_index_map=(0,)),
              (1,), mode=lax.GatherScatterMode.PROMISE_IN_BOUNDS)
          i = cidx * L + r
          for jj in range(K // L):
            x = gb[j, i, pl.ds(jj * L, L)]
            gb[j, i, pl.ds(jj * L, L)] = x * vb
      if do_gsq:
        def sq_grp(jj, _):
          v = va[g, pl.ds(jj * L, L)]
          sq[pl.ds(jj * L, L)] = v * v
          return 0
        lax.fori_loop(0, BATCH // L, sq_grp, 0)
        pltpu.sync_copy(sq, gsq_sh.at[si.at[g]], add=True)

    for j in range(ring - 1):
      fire_gather(j, j)

    def quad(q, _):
      for r in range(ring):
        g = q * ring + r
        wait_gather(g, r)
        scale(g, r)
        fire_scatter(g, r)
        j3 = (r + ring - 1) % ring
        if r == 0:
          @pl.when(q > 0)
          def _():
            wait_scatter(g - 1, j3)
          fire_gather(g + ring - 1, j3)
        else:
          @pl.when(q < nq - 1)
          def _():
            wait_scatter(g - 1, j3)
            fire_gather(g + ring - 1, j3)
      return 0
    lax.fori_loop(0, nq, quad, 0)

    for j in range(ring):
      wait_scatter(nb - ring + j, j)

    plsc.subcore_barrier()
    pltpu.sync_copy(acc_sh.at[pl.ds(s * RPT, RPT)],
                    acc_out.at[c].at[pl.ds(s * RPT, RPT)])
    if do_gsq:
      pltpu.sync_copy(gsq_sh.at[pl.ds(s * 1024, 1024)],
                      gsq_out.at[c].at[pl.ds(s * 1024, 1024)])

  return body


@functools.lru_cache(maxsize=None)
def _sc_pass(do_gsq):
  return _make_sc_pass(do_gsq)


def _sc_pred(vals, gidx, sidx, table):
  out = _sc_pass(False)(vals, gidx, sidx, table)
  return out[0] if isinstance(out, (list, tuple)) else out


def _sc_xtg(vals, gidx, sidx, table):
  acc, gsq = _sc_pass(True)(vals, gidx, sidx, table)
  return acc, gsq


def _prep_body(mb_ref, ph_ref, mz_ref, b_ref, inter_ref):
  i = pl.program_id(0)
  b_ref[...] = mb_ref[...] * ph_ref[...].T
  part = jnp.sum(mz_ref[...], axis=0, keepdims=True) * (1.0 / N)

  @pl.when(i == 0)
  def _():
    inter_ref[...] = part

  @pl.when(i > 0)
  def _():
    inter_ref[...] += part


def _prep(mean_beta, p_hat, mean_z):
  return pl.pallas_call(
      _prep_body,
      grid=(GRID,),
      in_specs=[
          pl.BlockSpec((PB, K), lambda i: (i, 0)),
          pl.BlockSpec((K, PB), lambda i: (0, i)),
          pl.BlockSpec((PB, K), lambda i: (i, 0)),
      ],
      out_specs=[
          pl.BlockSpec((PB, K), lambda i: (i, 0)),
          pl.BlockSpec((1, K), lambda i: (0, 0)),
      ],
      out_shape=[
          jax.ShapeDtypeStruct((P, K), jnp.float32),
          jax.ShapeDtypeStruct((1, K), jnp.float32),
      ],
  )(mean_beta, p_hat, mean_z)


def _x_body(mz_ref, pp_ref, inter_ref, x_ref):
  x_ref[...] = mz_ref[...] - pp_ref[0] - pp_ref[1] - inter_ref[...]


def _xcalc(mean_z, pred_part, inter):
  return pl.pallas_call(
      _x_body,
      grid=(GRID,),
      in_specs=[
          pl.BlockSpec((PB, K), lambda i: (i, 0)),
          pl.BlockSpec((NC, PB, K), lambda i: (0, i, 0)),
          pl.BlockSpec((1, K), lambda i: (0, 0)),
      ],
      out_specs=pl.BlockSpec((PB, K), lambda i: (i, 0)),
      out_shape=jax.ShapeDtypeStruct((N, K), jnp.float32),
  )(mean_z, pred_part, inter)


def _final_body(xtg_ref, b_ref, gsq_ref, tau_ref, lp_ref,
                mb_ref, vb_ref, ph_ref):
  gsq = (gsq_ref[0] + gsq_ref[1])[:, None]
  zkg = xtg_ref[0] + xtg_ref[1] + b_ref[...] * gsq
  var = 1.0 / (tau_ref[...] + gsq)
  mb = zkg * var
  ph = 1.0 / (1.0 + jnp.exp(-(lp_ref[0, 0] + 0.5 * mb * mb / var)))
  ph = jnp.clip(ph, 1e-8, 1.0 - 1e-8)
  mb_ref[...] = mb
  vb_ref[...] = var
  ph_ref[...] = ph.T


def _final(xtg_part, bmat, gsq_part, tau, lp):
  return pl.pallas_call(
      _final_body,
      grid=(GRID,),
      in_specs=[
          pl.BlockSpec((NC, PB, K), lambda i: (0, i, 0)),
          pl.BlockSpec((PB, K), lambda i: (i, 0)),
          pl.BlockSpec((NC, PB), lambda i: (0, i)),
          pl.BlockSpec((1, K), lambda i: (0, 0)),
          pl.BlockSpec((1, 1), lambda i: (0, 0)),
      ],
      out_specs=[
          pl.BlockSpec((PB, K), lambda i: (i, 0)),
          pl.BlockSpec((PB, K), lambda i: (i, 0)),
          pl.BlockSpec((K, PB), lambda i: (0, i)),
      ],
      out_shape=[
          jax.ShapeDtypeStruct((P, K), jnp.float32),
          jax.ShapeDtypeStruct((P, K), jnp.float32),
          jax.ShapeDtypeStruct((K, P), jnp.float32),
      ],
  )(xtg_part, bmat, gsq_part, tau, lp)


def kernel(guide_values, guide_rows, guide_cols, mean_z, mean_beta,
           var_beta, p_hat, tau_beta, p):
  pad = NPAD - NNZ

  def _shard(x):
    e0 = NS * NB0 * BATCH
    a = x[:e0].reshape(NS, NB0, BATCH)
    a = jnp.pad(a, ((0, 0), (0, NBMAX - NB0), (0, 0)))
    b = x[e0:].reshape(NS, NB1, BATCH)
    b = jnp.pad(b, ((0, 0), (0, NBMAX - NB1), (0, 0)))
    return jnp.concatenate([a, b], axis=0)

  vals = _shard(jnp.concatenate([guide_values, jnp.zeros((pad,),
                                                         jnp.float32)]))
  rows = _shard(jnp.concatenate([guide_rows,
                                 jnp.zeros((pad,), guide_rows.dtype)]))
  cols = _shard(jnp.concatenate([guide_cols,
                                 jnp.zeros((pad,), guide_cols.dtype)]))

  bmat, inter = _prep(mean_beta, p_hat, mean_z)
  pred_part = _sc_pred(vals, cols, rows, bmat)
  x = _xcalc(mean_z, pred_part, inter)
  xtg_part, gsq_part = _sc_xtg(vals, rows, cols, x)

  lp = (jnp.log(p) - jnp.log1p(-p)).reshape(1, 1).astype(jnp.float32)
  tau = tau_beta.reshape(1, K)
  return _final(xtg_part, bmat, gsq_part, tau, lp)


# speedup vs baseline: 1.2082x; 1.0080x over previous
import functools

import jax
import jax.numpy as jnp
from jax import lax
from jax.experimental import pallas as pl
from jax.experimental.pallas import tpu as pltpu
from jax.experimental.pallas import tpu_sc as plsc

N = 16384
P = 16384
K = 64
NNZ = 268435

NC = 2
NS = 16
L = 16
NW = NC * NS
BATCH = 128
NB0 = 78
NB1 = 54
NBMAX = max(NB0, NB1)
NPAD = NS * (NB0 + NB1) * BATCH
NB = NBMAX
XB = 32
RPT = 16384 // NS

PB = 512
GRID = 16384 // PB


def _make_sc_pass(do_x):
  ring = 3
  mesh = plsc.VectorSubcoreMesh(core_axis_name="c", subcore_axis_name="s",
                                num_cores=NC, num_subcores=NS)
  outs = [jax.ShapeDtypeStruct((NC, 16384, K), jnp.float32)]
  scratches = [
      pltpu.VMEM_SHARED((16384, K), jnp.float32),
      pltpu.VMEM((NB, BATCH), jnp.int32),
      pltpu.VMEM((NB, BATCH), jnp.int32),
      pltpu.VMEM((NB, BATCH), jnp.float32),
      pltpu.VMEM((ring, BATCH, K), jnp.float32),
      pltpu.VMEM((BATCH, K), jnp.float32),
  ] + [pltpu.SemaphoreType.DMA] * (2 * ring)
  if do_x:
    outs += [
        jax.ShapeDtypeStruct((NC, P), jnp.float32),
    ]
    scratches += [
        pltpu.VMEM_SHARED((P,), jnp.float32),
        pltpu.VMEM((1024,), jnp.float32),
        pltpu.VMEM((BATCH,), jnp.float32),
    ]

  @functools.partial(pl.kernel, out_type=outs, mesh=mesh,
                     scratch_types=scratches,
                     compiler_params=pltpu.CompilerParams(
                         use_tc_tiling_on_sc=False))
  def body(vals_h, gidx_h, sidx_h, *rest):
    table_h = rest[0]
    if do_x:
      acc_out, gsq_out = rest[1:3]
      acc_sh, gi, si, va, gb, z2 = rest[3:9]
      gsems, ssems = rest[9:9 + ring], rest[9 + ring:9 + 2 * ring]
      gsq_sh, z1, sq = rest[9 + 2 * ring:12 + 2 * ring]
    else:
      acc_out = rest[1]
      acc_sh, gi, si, va, gb, z2 = rest[2:8]
      gsems, ssems = rest[8:8 + ring], rest[8 + ring:8 + 2 * ring]
      gsq_out = gsq_sh = z1 = sq = None
    c = lax.axis_index("c")
    s = lax.axis_index("s")
    tid = c * NS + s
    nq = jnp.where(c == 0, NB0 // ring, NB1 // ring)
    nb = nq * ring
    do_gsq = do_x

    pltpu.sync_copy(gidx_h.at[tid], gi)
    pltpu.sync_copy(sidx_h.at[tid], si)
    pltpu.sync_copy(vals_h.at[tid], va)

    def zrow(i, _):
      for j in range(K // L):
        z2[i, pl.ds(j * L, L)] = jnp.zeros((L,), jnp.float32)
      return 0
    lax.fori_loop(0, BATCH, zrow, 0)
    for r in range(RPT // BATCH):
      pltpu.sync_copy(z2, acc_sh.at[pl.ds(s * RPT + r * BATCH, BATCH)])
    if do_gsq:
      def z1row(i, _):
        z1[pl.ds(i * L, L)] = jnp.zeros((L,), jnp.float32)
        return 0
      lax.fori_loop(0, 1024 // L, z1row, 0)
      pltpu.sync_copy(z1, gsq_sh.at[pl.ds(s * 1024, 1024)])
    plsc.subcore_barrier()

    def fire_gather(g, j):
      pltpu.async_copy(table_h.at[gi.at[g]], gb.at[j], gsems[j])

    def wait_gather(g, j):
      pltpu.make_async_copy(table_h.at[gi.at[g]], gb.at[j], gsems[j]).wait()

    def fire_scatter(g, j):
      pltpu.async_copy(gb.at[j], acc_sh.at[si.at[g]], ssems[j], add=True)

    def wait_scatter(g, j):
      pltpu.make_async_copy(gb.at[j], acc_sh.at[si.at[g]], ssems[j]).wait()

    def scale(g, j):
      @plsc.parallel_loop(0, BATCH // L, unroll=2)
      def _(cidx):
        vc = va[g, pl.ds(cidx * L, L)]
        for r in range(L):
          vb = lax.gather(
              vc, jnp.full((L, 1), r, jnp.int32),
              lax.GatherDimensionNumbers(offset_dims=(),
                                         collapsed_slice_dims=(0,),
                                         start_index_map=(0,)),
              (1,), mode=lax.GatherScatterMode.PROMISE_IN_BOUNDS)
          i = cidx * L + r
          for jj in range(K // L):
            x = gb[j, i, pl.ds(jj * L, L)]
            gb[j, i, pl.ds(jj * L, L)] = x * vb
      if do_gsq:
        def sq_grp(jj, _):
          v = va[g, pl.ds(jj * L, L)]
          sq[pl.ds(jj * L, L)] = v * v
          return 0
        lax.fori_loop(0, BATCH // L, sq_grp, 0)
        pltpu.sync_copy(sq, gsq_sh.at[si.at[g]], add=True)

    for j in range(ring - 1):
      fire_gather(j, j)

    def quad(q, _):
      for r in range(ring):
        g = q * ring + r
        wait_gather(g, r)
        scale(g, r)
        fire_scatter(g, r)
        j3 = (r + ring - 1) % ring
        if r == 0:
          @pl.when(q > 0)
          def _():
            wait_scatter(g - 1, j3)
          fire_gather(g + ring - 1, j3)
        else:
          @pl.when(q < nq - 1)
          def _():
            wait_scatter(g - 1, j3)
            fire_gather(g + ring - 1, j3)
      return 0
    lax.fori_loop(0, nq, quad, 0)

    for j in range(ring):
      wait_scatter(nb - ring + j, j)

    plsc.subcore_barrier()
    pltpu.sync_copy(acc_sh.at[pl.ds(s * RPT, RPT)],
                    acc_out.at[c].at[pl.ds(s * RPT, RPT)])
    if do_gsq:
      pltpu.sync_copy(gsq_sh.at[pl.ds(s * 1024, 1024)],
                      gsq_out.at[c].at[pl.ds(s * 1024, 1024)])

  return body


@functools.lru_cache(maxsize=None)
def _sc_pass(do_gsq):
  return _make_sc_pass(do_gsq)


def _sc_pred(vals, gidx, sidx, table):
  out = _sc_pass(False)(vals, gidx, sidx, table)
  return out[0] if isinstance(out, (list, tuple)) else out


def _sc_xtg(vals, gidx, sidx, table):
  acc, gsq = _sc_pass(True)(vals, gidx, sidx, table)
  return acc, gsq


def _prep_body(mb_ref, ph_ref, mz_ref, b_ref, inter_ref):
  i = pl.program_id(0)
  b_ref[...] = mb_ref[...] * ph_ref[...].T
  part = jnp.sum(mz_ref[...], axis=0, keepdims=True) * (1.0 / N)

  @pl.when(i == 0)
  def _():
    inter_ref[...] = part

  @pl.when(i > 0)
  def _():
    inter_ref[...] += part


def _prep(mean_beta, p_hat, mean_z):
  return pl.pallas_call(
      _prep_body,
      grid=(GRID,),
      in_specs=[
          pl.BlockSpec((PB, K), lambda i: (i, 0)),
          pl.BlockSpec((K, PB), lambda i: (0, i)),
          pl.BlockSpec((PB, K), lambda i: (i, 0)),
      ],
      out_specs=[
          pl.BlockSpec((PB, K), lambda i: (i, 0)),
          pl.BlockSpec((1, K), lambda i: (0, 0)),
      ],
      out_shape=[
          jax.ShapeDtypeStruct((P, K), jnp.float32),
          jax.ShapeDtypeStruct((1, K), jnp.float32),
      ],
  )(mean_beta, p_hat, mean_z)


def _x_body(mz_ref, pp_ref, inter_ref, x_ref):
  x_ref[...] = mz_ref[...] - pp_ref[0] - pp_ref[1] - inter_ref[...]


def _xcalc(mean_z, pred_part, inter):
  return pl.pallas_call(
      _x_body,
      grid=(GRID,),
      in_specs=[
          pl.BlockSpec((PB, K), lambda i: (i, 0)),
          pl.BlockSpec((NC, PB, K), lambda i: (0, i, 0)),
          pl.BlockSpec((1, K), lambda i: (0, 0)),
      ],
      out_specs=pl.BlockSpec((PB, K), lambda i: (i, 0)),
      out_shape=jax.ShapeDtypeStruct((N, K), jnp.float32),
  )(mean_z, pred_part, inter)


def _final_body(xtg_ref, b_ref, gsq_ref, tau_ref, lp_ref,
                mb_ref, vb_ref, ph_ref):
  gsq = (gsq_ref[0] + gsq_ref[1])[:, None]
  zkg = xtg_ref[0] + xtg_ref[1] + b_ref[...] * gsq
  var = 1.0 / (tau_ref[...] + gsq)
  mb = zkg * var
  ph = 1.0 / (1.0 + jnp.exp(-(lp_ref[0, 0] + 0.5 * mb * mb / var)))
  ph = jnp.clip(ph, 1e-8, 1.0 - 1e-8)
  mb_ref[...] = mb
  vb_ref[...] = var
  ph_ref[...] = ph.T


def _final(xtg_part, bmat, gsq_part, tau, lp):
  return pl.pallas_call(
      _final_body,
      grid=(GRID,),
      in_specs=[
          pl.BlockSpec((NC, PB, K), lambda i: (0, i, 0)),
          pl.BlockSpec((PB, K), lambda i: (i, 0)),
          pl.BlockSpec((NC, PB), lambda i: (0, i)),
          pl.BlockSpec((1, K), lambda i: (0, 0)),
          pl.BlockSpec((1, 1), lambda i: (0, 0)),
      ],
      out_specs=[
          pl.BlockSpec((PB, K), lambda i: (i, 0)),
          pl.BlockSpec((PB, K), lambda i: (i, 0)),
          pl.BlockSpec((K, PB), lambda i: (0, i)),
      ],
      out_shape=[
          jax.ShapeDtypeStruct((P, K), jnp.float32),
          jax.ShapeDtypeStruct((P, K), jnp.float32),
          jax.ShapeDtypeStruct((K, P), jnp.float32),
      ],
  )(xtg_part, bmat, gsq_part, tau, lp)


def kernel(guide_values, guide_rows, guide_cols, mean_z, mean_beta,
           var_beta, p_hat, tau_beta, p):
  pad = NPAD - NNZ

  def _shard(x):
    e0 = NS * NB0 * BATCH
    a = x[:e0].reshape(NS, NB0, BATCH)
    a = jnp.pad(a, ((0, 0), (0, NBMAX - NB0), (0, 0)))
    b = x[e0:].reshape(NS, NB1, BATCH)
    b = jnp.pad(b, ((0, 0), (0, NBMAX - NB1), (0, 0)))
    return jnp.concatenate([a, b], axis=0)

  vals = _shard(jnp.concatenate([guide_values, jnp.zeros((pad,),
                                                         jnp.float32)]))
  rows = _shard(jnp.concatenate([guide_rows,
                                 jnp.zeros((pad,), guide_rows.dtype)]))
  cols = _shard(jnp.concatenate([guide_cols,
                                 jnp.zeros((pad,), guide_cols.dtype)]))

  bmat, inter = _prep(mean_beta, p_hat, mean_z)
  pred_part = _sc_pred(vals, cols, rows, bmat)
  x = _xcalc(mean_z, pred_part, inter)
  xtg_part, gsq_part = _sc_xtg(vals, rows, cols, x)

  lp = (jnp.log(p) - jnp.log1p(-p)).reshape(1, 1).astype(jnp.float32)
  tau = tau_beta.reshape(1, K)
  return _final(xtg_part, bmat, gsq_part, tau, lp)


# speedup vs baseline: 1.2139x; 1.0047x over previous
import functools

import jax
import jax.numpy as jnp
from jax import lax
from jax.experimental import pallas as pl
from jax.experimental.pallas import tpu as pltpu
from jax.experimental.pallas import tpu_sc as plsc

N = 16384
P = 16384
K = 64
NNZ = 268435

NC = 2
NS = 16
L = 16
NW = NC * NS
BATCH = 128
NB0 = 78
NB1 = 54
NBMAX = max(NB0, NB1)
NPAD = NS * (NB0 + NB1) * BATCH
NB = NBMAX
RPT = 16384 // NS

PB = 512
GRID = 16384 // PB


def _make_sc_pass(do_x):
  ring = 3
  mesh = plsc.VectorSubcoreMesh(core_axis_name="c", subcore_axis_name="s",
                                num_cores=NC, num_subcores=NS)
  outs = [jax.ShapeDtypeStruct((NC, 16384, K), jnp.float32)]
  scratches = [
      pltpu.VMEM_SHARED((16384, K), jnp.float32),
      pltpu.VMEM((NB, BATCH), jnp.int32),
      pltpu.VMEM((NB, BATCH), jnp.int32),
      pltpu.VMEM((NB, BATCH), jnp.float32),
      pltpu.VMEM((ring, BATCH, K), jnp.float32),
      pltpu.VMEM((BATCH, K), jnp.float32),
  ] + [pltpu.SemaphoreType.DMA] * (2 * ring)
  if do_x:
    outs += [
        jax.ShapeDtypeStruct((NC, P), jnp.float32),
    ]
    scratches += [
        pltpu.VMEM_SHARED((P,), jnp.float32),
        pltpu.VMEM((1024,), jnp.float32),
        pltpu.VMEM((BATCH,), jnp.float32),
    ]

  @functools.partial(pl.kernel, out_type=outs, mesh=mesh,
                     scratch_types=scratches,
                     compiler_params=pltpu.CompilerParams(
                         use_tc_tiling_on_sc=False))
  def body(vals_h, gidx_h, sidx_h, *rest):
    table_h = rest[0]
    if do_x:
      acc_out, gsq_out = rest[1:3]
      acc_sh, gi, si, va, gb, z2 = rest[3:9]
      gsems, ssems = rest[9:9 + ring], rest[9 + ring:9 + 2 * ring]
      gsq_sh, z1, sq = rest[9 + 2 * ring:12 + 2 * ring]
    else:
      acc_out = rest[1]
      acc_sh, gi, si, va, gb, z2 = rest[2:8]
      gsems, ssems = rest[8:8 + ring], rest[8 + ring:8 + 2 * ring]
      gsq_out = gsq_sh = z1 = sq = None
    c = lax.axis_index("c")
    s = lax.axis_index("s")
    tid = c * NS + s
    nq = jnp.where(c == 0, NB0 // ring, NB1 // ring)
    nb = nq * ring
    do_gsq = do_x

    pltpu.sync_copy(gidx_h.at[tid], gi)
    pltpu.sync_copy(sidx_h.at[tid], si)
    pltpu.sync_copy(vals_h.at[tid], va)

    def zrow(i, _):
      for j in range(K // L):
        z2[i, pl.ds(j * L, L)] = jnp.zeros((L,), jnp.float32)
      return 0
    lax.fori_loop(0, BATCH, zrow, 0)
    for r in range(RPT // BATCH):
      pltpu.sync_copy(z2, acc_sh.at[pl.ds(s * RPT + r * BATCH, BATCH)])
    if do_gsq:
      def z1row(i, _):
        z1[pl.ds(i * L, L)] = jnp.zeros((L,), jnp.float32)
        return 0
      lax.fori_loop(0, 1024 // L, z1row, 0)
      pltpu.sync_copy(z1, gsq_sh.at[pl.ds(s * 1024, 1024)])
    plsc.subcore_barrier()

    def fire_gather(g, j):
      pltpu.async_copy(table_h.at[gi.at[g]], gb.at[j], gsems[j])

    def wait_gather(g, j):
      pltpu.make_async_copy(table_h.at[gi.at[g]], gb.at[j], gsems[j]).wait()

    def fire_scatter(g, j):
      pltpu.async_copy(gb.at[j], acc_sh.at[si.at[g]], ssems[j], add=True)

    def wait_scatter(g, j):
      pltpu.make_async_copy(gb.at[j], acc_sh.at[si.at[g]], ssems[j]).wait()

    def scale(g, j):
      @plsc.parallel_loop(0, BATCH // L, unroll=2)
      def _(cidx):
        vc = va[g, pl.ds(cidx * L, L)]
        for r in range(L):
          vb = lax.gather(
              vc, jnp.full((L, 1), r, jnp.int32),
              lax.GatherDimensionNumbers(offset_dims=(),
                                         collapsed_slice_dims=(0,),
                                         start_index_map=(0,)),
              (1,), mode=lax.GatherScatterMode.PROMISE_IN_BOUNDS)
          i = cidx * L + r
          for jj in range(K // L):
            x = gb[j, i, pl.ds(jj * L, L)]
            gb[j, i, pl.ds(jj * L, L)] = x * vb
      if do_gsq:
        def sq_grp(jj, _):
          v = va[g, pl.ds(jj * L, L)]
          sq[pl.ds(jj * L, L)] = v * v
          return 0
        lax.fori_loop(0, BATCH // L, sq_grp, 0)
        pltpu.sync_copy(sq, gsq_sh.at[si.at[g]], add=True)

    for j in range(ring - 1):
      fire_gather(j, j)

    def quad(q, _):
      for r in range(ring):
        g = q * ring + r
        wait_gather(g, r)
        scale(g, r)
        fire_scatter(g, r)
        j3 = (r + ring - 1) % ring
        if r == 0:
          @pl.when(q > 0)
          def _():
            wait_scatter(g - 1, j3)
          fire_gather(g + ring - 1, j3)
        else:
          @pl.when(q < nq - 1)
          def _():
            wait_scatter(g - 1, j3)
            fire_gather(g + ring - 1, j3)
      return 0
    lax.fori_loop(0, nq, quad, 0)

    for j in range(ring):
      wait_scatter(nb - ring + j, j)

    plsc.subcore_barrier()
    pltpu.sync_copy(acc_sh.at[pl.ds(s * RPT, RPT)],
                    acc_out.at[c].at[pl.ds(s * RPT, RPT)])
    if do_gsq:
      pltpu.sync_copy(gsq_sh.at[pl.ds(s * 1024, 1024)],
                      gsq_out.at[c].at[pl.ds(s * 1024, 1024)])

  return body


@functools.lru_cache(maxsize=None)
def _sc_pass(do_gsq):
  return _make_sc_pass(do_gsq)


def _sc_pred(vals, gidx, sidx, table):
  out = _sc_pass(False)(vals, gidx, sidx, table)
  return out[0] if isinstance(out, (list, tuple)) else out


def _sc_xtg(vals, gidx, sidx, table):
  acc, gsq = _sc_pass(True)(vals, gidx, sidx, table)
  return acc, gsq


def _prep_body(mb_ref, ph_ref, mz_ref, b_ref, inter_ref):
  i = pl.program_id(0)
  b_ref[...] = mb_ref[...] * ph_ref[...].T
  part = jnp.sum(mz_ref[...], axis=0, keepdims=True) * (1.0 / N)

  @pl.when(i == 0)
  def _():
    inter_ref[...] = part

  @pl.when(i > 0)
  def _():
    inter_ref[...] += part


def _prep(mean_beta, p_hat, mean_z):
  return pl.pallas_call(
      _prep_body,
      grid=(GRID,),
      in_specs=[
          pl.BlockSpec((PB, K), lambda i: (i, 0)),
          pl.BlockSpec((K, PB), lambda i: (0, i)),
          pl.BlockSpec((PB, K), lambda i: (i, 0)),
      ],
      out_specs=[
          pl.BlockSpec((PB, K), lambda i: (i, 0)),
          pl.BlockSpec((1, K), lambda i: (0, 0)),
      ],
      out_shape=[
          jax.ShapeDtypeStruct((P, K), jnp.float32),
          jax.ShapeDtypeStruct((1, K), jnp.float32),
      ],
  )(mean_beta, p_hat, mean_z)


def _x_body(mz_ref, pp_ref, inter_ref, x_ref):
  x_ref[...] = mz_ref[...] - pp_ref[0] - pp_ref[1] - inter_ref[...]


def _xcalc(mean_z, pred_part, inter):
  return pl.pallas_call(
      _x_body,
      grid=(GRID,),
      in_specs=[
          pl.BlockSpec((PB, K), lambda i: (i, 0)),
          pl.BlockSpec((NC, PB, K), lambda i: (0, i, 0)),
          pl.BlockSpec((1, K), lambda i: (0, 0)),
      ],
      out_specs=pl.BlockSpec((PB, K), lambda i: (i, 0)),
      out_shape=jax.ShapeDtypeStruct((N, K), jnp.float32),
  )(mean_z, pred_part, inter)


def _final_body(xtg_ref, b_ref, gsq_ref, tau_ref, lp_ref,
                mb_ref, vb_ref, ph_ref):
  gsq = (gsq_ref[0] + gsq_ref[1])[:, None]
  zkg = xtg_ref[0] + xtg_ref[1] + b_ref[...] * gsq
  var = 1.0 / (tau_ref[...] + gsq)
  mb = zkg * var
  ph = 1.0 / (1.0 + jnp.exp(-(lp_ref[0, 0] + 0.5 * mb * mb / var)))
  ph = jnp.clip(ph, 1e-8, 1.0 - 1e-8)
  mb_ref[...] = mb
  vb_ref[...] = var
  ph_ref[...] = ph.T


def _final(xtg_part, bmat, gsq_part, tau, lp):
  return pl.pallas_call(
      _final_body,
      grid=(GRID,),
      in_specs=[
          pl.BlockSpec((NC, PB, K), lambda i: (0, i, 0)),
          pl.BlockSpec((PB, K), lambda i: (i, 0)),
          pl.BlockSpec((NC, PB), lambda i: (0, i)),
          pl.BlockSpec((1, K), lambda i: (0, 0)),
          pl.BlockSpec((1, 1), lambda i: (0, 0)),
      ],
      out_specs=[
          pl.BlockSpec((PB, K), lambda i: (i, 0)),
          pl.BlockSpec((PB, K), lambda i: (i, 0)),
          pl.BlockSpec((K, PB), lambda i: (0, i)),
      ],
      out_shape=[
          jax.ShapeDtypeStruct((P, K), jnp.float32),
          jax.ShapeDtypeStruct((P, K), jnp.float32),
          jax.ShapeDtypeStruct((K, P), jnp.float32),
      ],
  )(xtg_part, bmat, gsq_part, tau, lp)


def kernel(guide_values, guide_rows, guide_cols, mean_z, mean_beta,
           var_beta, p_hat, tau_beta, p):
  pad = NPAD - NNZ

  def _shard(x):
    e0 = NS * NB0 * BATCH
    a = x[:e0].reshape(NS, NB0, BATCH)
    a = jnp.pad(a, ((0, 0), (0, NBMAX - NB0), (0, 0)))
    b = x[e0:].reshape(NS, NB1, BATCH)
    b = jnp.pad(b, ((0, 0), (0, NBMAX - NB1), (0, 0)))
    return jnp.concatenate([a, b], axis=0)

  vals = _shard(jnp.concatenate([guide_values, jnp.zeros((pad,),
                                                         jnp.float32)]))
  rows = _shard(jnp.concatenate([guide_rows,
                                 jnp.zeros((pad,), guide_rows.dtype)]))
  cols = _shard(jnp.concatenate([guide_cols,
                                 jnp.zeros((pad,), guide_cols.dtype)]))

  bmat, inter = _prep(mean_beta, p_hat, mean_z)
  pred_part = _sc_pred(vals, cols, rows, bmat)
  x = _xcalc(mean_z, pred_part, inter)
  xtg_part, gsq_part = _sc_xtg(vals, rows, cols, x)

  lp = (jnp.log(p) - jnp.log1p(-p)).reshape(1, 1).astype(jnp.float32)
  tau = tau_beta.reshape(1, K)
  return _final(xtg_part, bmat, gsq_part, tau, lp)
